# Initial kernel scaffold; baseline (speedup 1.0000x reference)
import jax
import jax.numpy as jnp
from jax.experimental import pallas as pl


def kernel(query_vectors, temperature, indices, sofa_metric):
    raise NotImplementedError("write your pallas kernel here")


# speedup vs baseline: 1.3565x; 1.3565x over previous
import jax
import jax.numpy as jnp
from jax.experimental import pallas as pl
from jax.experimental.pallas import tpu as pltpu

_EPS = 1e-8
_BQ = 256


def _body(t_ref, ix_ref, iy_ref, s_ref, qx_ref, qy_ref, o_ref):
    t = t_ref[0, 0]
    s = s_ref[...]
    m = (s - jnp.min(s)) / (jnp.max(s) - jnp.min(s))
    qx = qx_ref[...]
    qy = qy_ref[...]
    ix = ix_ref[...]
    iy = iy_ref[...]
    q_norm = qx * qx + qy * qy
    i_norm = ix * ix + iy * iy
    qxb = qx.astype(jnp.bfloat16).astype(jnp.float32)
    qyb = qy.astype(jnp.bfloat16).astype(jnp.float32)
    ixb = ix.astype(jnp.bfloat16).astype(jnp.float32)
    iyb = iy.astype(jnp.bfloat16).astype(jnp.float32)
    dot = qxb * ixb + qyb * iyb
    dists = q_norm + i_norm - 2 * dot
    x = -dists / (t + _EPS)
    x_max = jnp.max(x, axis=1, keepdims=True)
    e = jnp.exp(x - x_max)
    z = jnp.sum(e, axis=1, keepdims=True)
    w = e / z
    w = jnp.where(w < 0.001, 0.0, w)
    o_ref[...] = jnp.sum(w * m, axis=1, keepdims=True)


def kernel(query_vectors, temperature, indices, sofa_metric):
    batch, _ = query_vectors.shape
    db, _ = indices.shape
    orig_dtype = query_vectors.dtype
    q = query_vectors.astype(jnp.float32)
    qx = q[:, 0:1]
    qy = q[:, 1:2]
    ind = indices.astype(jnp.float32)
    ix = ind[:, 0].reshape(1, db)
    iy = ind[:, 1].reshape(1, db)
    s = sofa_metric.astype(jnp.float32).reshape(1, db)
    t = temperature.astype(jnp.float32).reshape(1, 1)

    grid = batch // _BQ
    out = pl.pallas_call(
        _body,
        grid=(grid,),
        in_specs=[
            pl.BlockSpec((1, 1), lambda i: (0, 0)),
            pl.BlockSpec((1, db), lambda i: (0, 0)),
            pl.BlockSpec((1, db), lambda i: (0, 0)),
            pl.BlockSpec((1, db), lambda i: (0, 0)),
            pl.BlockSpec((_BQ, 1), lambda i: (i, 0)),
            pl.BlockSpec((_BQ, 1), lambda i: (i, 0)),
        ],
        out_specs=pl.BlockSpec((_BQ, 1), lambda i: (i, 0)),
        out_shape=jax.ShapeDtypeStruct((batch, 1), jnp.float32),
        compiler_params=pltpu.CompilerParams(
            dimension_semantics=("arbitrary",),
        ),
    )(t, ix, iy, s, qx, qy)
    return out.reshape(batch).astype(orig_dtype)


# speedup vs baseline: 1.6797x; 1.2383x over previous
import jax
import jax.numpy as jnp
from jax.experimental import pallas as pl
from jax.experimental.pallas import tpu as pltpu

_EPS = 1e-8
_BQ = 256


def _body(t_ref, ix_ref, iy_ref, s_ref, qx_ref, qy_ref, o_ref):
    inv_t = 1.0 / (t_ref[0, 0] + _EPS)
    s = s_ref[...]
    m = (s - jnp.min(s)) / (jnp.max(s) - jnp.min(s))
    ix = ix_ref[...]
    iy = iy_ref[...]
    gi = (ix * ix + iy * iy) * inv_t
    ixb = ix.astype(jnp.bfloat16).astype(jnp.float32)
    iyb = iy.astype(jnp.bfloat16).astype(jnp.float32)
    qxb = qx_ref[...].astype(jnp.bfloat16).astype(jnp.float32)
    qyb = qy_ref[...].astype(jnp.bfloat16).astype(jnp.float32)
    dot = qxb * ixb + qyb * iyb
    u = dot * (2.0 * inv_t) - gi
    umax = jnp.max(u, axis=1, keepdims=True)
    e = jnp.exp(u - umax)
    z = jnp.sum(e, axis=1, keepdims=True)
    num = jnp.sum(jnp.where(e >= 0.001 * z, e * m, 0.0), axis=1, keepdims=True)
    o_ref[...] = num / z


def kernel(query_vectors, temperature, indices, sofa_metric):
    batch, _ = query_vectors.shape
    db, _ = indices.shape
    orig_dtype = query_vectors.dtype
    q = query_vectors.astype(jnp.float32)
    qx = q[:, 0:1]
    qy = q[:, 1:2]
    ind = indices.astype(jnp.float32)
    ix = ind[:, 0].reshape(1, db)
    iy = ind[:, 1].reshape(1, db)
    s = sofa_metric.astype(jnp.float32).reshape(1, db)
    t = temperature.astype(jnp.float32).reshape(1, 1)

    grid = batch // _BQ
    full = lambda i: (0, 0)
    rows = lambda i: (i, 0)
    out = pl.pallas_call(
        _body,
        grid=(grid,),
        in_specs=[
            pl.BlockSpec((1, 1), full),
            pl.BlockSpec((1, db), full),
            pl.BlockSpec((1, db), full),
            pl.BlockSpec((1, db), full),
            pl.BlockSpec((_BQ, 1), rows),
            pl.BlockSpec((_BQ, 1), rows),
        ],
        out_specs=pl.BlockSpec((_BQ, 1), rows),
        out_shape=jax.ShapeDtypeStruct((batch, 1), jnp.float32),
        compiler_params=pltpu.CompilerParams(
            dimension_semantics=("arbitrary",),
        ),
    )(t, ix, iy, s, qx, qy)
    return out.reshape(batch).astype(orig_dtype)
